# baseline (speedup 1.0000x reference)
import jax
import jax.numpy as jnp
from jax.experimental import pallas as pl
from jax.experimental.pallas import tpu as pltpu


def _gat_body(x_ref, adj_ref, w_ref, a_ref, o_ref, hc_ref, hcb_ref):
    _, g, n, fin = x_ref.shape
    fout = w_ref.shape[1]
    x = x_ref[0].reshape(g * n, fin)
    h = jnp.dot(x, w_ref[...], preferred_element_type=jnp.float32)
    h3 = h.reshape(g, n, fout)
    for k in range(g):
        hc_ref[:, k * fout:(k + 1) * fout] = h3[k]
        hcb_ref[:, k * fout:(k + 1) * fout] = h3[k].astype(jnp.bfloat16)

    mask = (adj_ref[...] > 0).astype(jnp.bfloat16)
    h2c = jax.lax.dot_general(
        mask, hcb_ref[...], (((0,), (0,)), ((), ())),
        preferred_element_type=jnp.float32,
    )
    a1t = a_ref[:fout, :].T
    a2t = a_ref[fout:, :].T
    a1rep = jnp.concatenate([a1t] * g, axis=1)
    a2rep = jnp.concatenate([a2t] * g, axis=1)
    hcf = hc_ref[...]
    p = hcf * a1rep + h2c * a2rep
    rows = jax.lax.broadcasted_iota(jnp.int32, (g * fout, g), 0)
    cols = jax.lax.broadcasted_iota(jnp.int32, (g * fout, g), 1)
    bsel = (rows // fout == cols).astype(jnp.float32)
    scores = jnp.dot(p, bsel, preferred_element_type=jnp.float32)
    hsums = jnp.sum(hcf, axis=0, keepdims=True)
    for k in range(g):
        sc = jnp.broadcast_to(scores[:, k:k + 1], (n, fout))
        hs = jnp.broadcast_to(hsums[:, k * fout:(k + 1) * fout], (n, fout))
        o_ref[0, k] = jnp.maximum(sc * hs, 0.0)


def kernel(inp, adj, W, a):
    b, t, n, fin = inp.shape
    fout = W.shape[1]
    inp = inp.astype(jnp.bfloat16)
    W = W.astype(jnp.bfloat16)

    return pl.pallas_call(
        _gat_body,
        grid=(b,),
        in_specs=[
            pl.BlockSpec((1, t, n, fin), lambda i: (i, 0, 0, 0)),
            pl.BlockSpec((n, n), lambda i: (0, 0)),
            pl.BlockSpec((fin, fout), lambda i: (0, 0)),
            pl.BlockSpec((2 * fout, n), lambda i: (0, 0)),
        ],
        out_specs=pl.BlockSpec((1, t, n, fout), lambda i: (i, 0, 0, 0)),
        out_shape=jax.ShapeDtypeStruct((b, t, n, fout), jnp.float32),
        scratch_shapes=[
            pltpu.VMEM((n, t * fout), jnp.float32),
            pltpu.VMEM((n, t * fout), jnp.bfloat16),
        ],
        compiler_params=pltpu.CompilerParams(
            dimension_semantics=(pltpu.PARALLEL,),
        ),
    )(inp, adj, W, a)


# speedup vs baseline: 1.2920x; 1.2920x over previous
import jax
import jax.numpy as jnp
from jax.experimental import pallas as pl
from jax.experimental.pallas import tpu as pltpu


def _gat_body(x_hbm, adj_ref, wt_ref, a_ref, o_ref,
              xbuf, hc_ref, hcb_ref, sems):
    i = pl.program_id(0)
    nb = pl.num_programs(0)
    _, g, n, fin = x_hbm.shape
    fout = wt_ref.shape[0]

    def start(blk, slot):
        pltpu.make_async_copy(
            x_hbm.at[blk], xbuf.at[slot], sems.at[slot]
        ).start()

    def wait(blk, slot):
        pltpu.make_async_copy(
            x_hbm.at[blk], xbuf.at[slot], sems.at[slot]
        ).wait()

    @pl.when(i == 0)
    def _():
        start(0, 0)

    @pl.when(i + 1 < nb)
    def _():
        start(i + 1, (i + 1) % 2)

    wait(i, i % 2)
    slot = i % 2
    x = xbuf[slot].reshape(g * n, fin)
    h = jax.lax.dot_general(
        x, wt_ref[...], (((1,), (1,)), ((), ())),
        preferred_element_type=jnp.float32,
    )
    h3 = h.reshape(g, n, fout)
    for k in range(g):
        hc_ref[:, k * fout:(k + 1) * fout] = h3[k]
        hcb_ref[:, k * fout:(k + 1) * fout] = h3[k].astype(jnp.bfloat16)
    mask = (adj_ref[...] > 0).astype(jnp.bfloat16)
    h2c = jax.lax.dot_general(
        mask, hcb_ref[...], (((0,), (0,)), ((), ())),
        preferred_element_type=jnp.float32,
    )
    a1t = a_ref[:fout, :].T
    a2t = a_ref[fout:, :].T
    a1rep = jnp.concatenate([a1t] * g, axis=1)
    a2rep = jnp.concatenate([a2t] * g, axis=1)
    hcf = hc_ref[...]
    p = hcf * a1rep + h2c * a2rep
    rows = jax.lax.broadcasted_iota(jnp.int32, (g * fout, g), 0)
    cols = jax.lax.broadcasted_iota(jnp.int32, (g * fout, g), 1)
    bsel = (rows // fout == cols).astype(jnp.float32)
    scores = jnp.dot(p, bsel, preferred_element_type=jnp.float32)
    hsums = jnp.sum(hcf, axis=0, keepdims=True)
    for k in range(g):
        sc = jnp.broadcast_to(scores[:, k:k + 1], (n, fout))
        hs = jnp.broadcast_to(hsums[:, k * fout:(k + 1) * fout], (n, fout))
        o_ref[0, k] = jnp.maximum(sc * hs, 0.0)


def kernel(inp, adj, W, a):
    b, t, n, fin = inp.shape
    fout = W.shape[1]

    return pl.pallas_call(
        _gat_body,
        grid=(b,),
        in_specs=[
            pl.BlockSpec(memory_space=pltpu.MemorySpace.HBM),
            pl.BlockSpec((n, n), lambda i: (0, 0)),
            pl.BlockSpec((fout, fin), lambda i: (0, 0)),
            pl.BlockSpec((2 * fout, n), lambda i: (0, 0)),
        ],
        out_specs=pl.BlockSpec((1, t, n, fout), lambda i: (i, 0, 0, 0)),
        out_shape=jax.ShapeDtypeStruct((b, t, n, fout), jnp.float32),
        scratch_shapes=[
            pltpu.VMEM((2, t, n, fin), jnp.float32),
            pltpu.VMEM((n, t * fout), jnp.float32),
            pltpu.VMEM((n, t * fout), jnp.bfloat16),
            pltpu.SemaphoreType.DMA((2,)),
        ],
        compiler_params=pltpu.CompilerParams(
            dimension_semantics=(pltpu.ARBITRARY,),
        ),
    )(inp, adj, W.T, a)


# speedup vs baseline: 2.2437x; 1.7366x over previous
import jax
import jax.numpy as jnp
from jax.experimental import pallas as pl
from jax.experimental.pallas import tpu as pltpu


def _gat_body(x_hbm, adj_ref, wt_ref, a_ref, o_ref,
              xbuf, hc_ref, hcb_ref, sems):
    i = pl.program_id(0)
    nb = pl.num_programs(0)
    _, g, n, fin = x_hbm.shape
    fout = wt_ref.shape[0]

    def start(blk, slot):
        pltpu.make_async_copy(
            x_hbm.at[blk], xbuf.at[slot], sems.at[slot]
        ).start()

    def wait(blk, slot):
        pltpu.make_async_copy(
            x_hbm.at[blk], xbuf.at[slot], sems.at[slot]
        ).wait()

    @pl.when(i == 0)
    def _():
        start(0, 0)

    @pl.when(i + 1 < nb)
    def _():
        start(i + 1, (i + 1) % 2)

    wait(i, i % 2)
    slot = i % 2
    x = xbuf[slot].reshape(g * n, fin)
    h = jax.lax.dot_general(
        x, wt_ref[...], (((1,), (1,)), ((), ())),
        preferred_element_type=jnp.float32,
    )
    h3 = h.reshape(g, n, fout)
    for k in range(g):
        hc_ref[:, k * fout:(k + 1) * fout] = h3[k]
        hcb_ref[:, k * fout:(k + 1) * fout] = h3[k].astype(jnp.bfloat16)
    mask = (adj_ref[...] > 0).astype(jnp.bfloat16)
    h2c = jax.lax.dot_general(
        mask, hcb_ref[...], (((0,), (0,)), ((), ())),
        preferred_element_type=jnp.float32,
    )
    a1t = a_ref[:fout, :].T
    a2t = a_ref[fout:, :].T
    a1rep = jnp.concatenate([a1t] * g, axis=1)
    a2rep = jnp.concatenate([a2t] * g, axis=1)
    hcf = hc_ref[...]
    p = hcf * a1rep + h2c * a2rep
    rows = jax.lax.broadcasted_iota(jnp.int32, (g * fout, g), 0)
    cols = jax.lax.broadcasted_iota(jnp.int32, (g * fout, g), 1)
    bsel = (rows // fout == cols).astype(jnp.float32)
    scores = jnp.dot(p, bsel, preferred_element_type=jnp.float32)
    hsums = jnp.sum(hcf, axis=0, keepdims=True)
    scT = scores.T
    hsT = hsums.T
    for k in range(g):
        sc = jnp.broadcast_to(scT[k:k + 1, :], (fout, n))
        hs = jnp.broadcast_to(hsT[k * fout:(k + 1) * fout, :], (fout, n))
        o_ref[0, k] = jnp.maximum(sc * hs, 0.0)


def kernel(inp, adj, W, a):
    b, t, n, fin = inp.shape
    fout = W.shape[1]

    out = pl.pallas_call(
        _gat_body,
        grid=(b,),
        in_specs=[
            pl.BlockSpec(memory_space=pltpu.MemorySpace.HBM),
            pl.BlockSpec((n, n), lambda i: (0, 0)),
            pl.BlockSpec((fout, fin), lambda i: (0, 0)),
            pl.BlockSpec((2 * fout, n), lambda i: (0, 0)),
        ],
        out_specs=pl.BlockSpec((1, t, fout, n), lambda i: (i, 0, 0, 0)),
        out_shape=jax.ShapeDtypeStruct((b, t, fout, n), jnp.float32),
        scratch_shapes=[
            pltpu.VMEM((2, t, n, fin), jnp.float32),
            pltpu.VMEM((n, t * fout), jnp.float32),
            pltpu.VMEM((n, t * fout), jnp.bfloat16),
            pltpu.SemaphoreType.DMA((2,)),
        ],
        compiler_params=pltpu.CompilerParams(
            dimension_semantics=(pltpu.ARBITRARY,),
        ),
    )(inp, adj, W.T, a)
    return out.swapaxes(2, 3)
